# baseline (speedup 1.0000x reference)
import functools
import math

import jax
import jax.numpy as jnp
from jax import lax
from jax.experimental import pallas as pl
from jax.experimental.pallas import tpu as pltpu
from jax.experimental.pallas import tpu_sc as plsc

_D = 1024
_LANES = 16
_NC = 2
_NS = 16
_NW = _NC * _NS
_SCALE = math.sqrt(_D)
_CHUNK = 8


def _emb_body(x_hbm, wtab_hbm, ptab_hbm, out_hbm,
              idx_all, posbuf, w0, w1, ob0, ob1,
              g0, g1, o0, o1, psem,
              *, batch, seq):
    wid = lax.axis_index("s") * _NC + lax.axis_index("c")
    pos_per_w = seq // _NW
    kmax = pos_per_w // _CHUNK
    n_chunks = kmax * batch
    jcols = _D // _LANES
    xrows_per_b = seq // _CHUNK

    w = [w0, w1]
    ob = [ob0, ob1]
    gsem = [g0, g1]
    osem = [o0, o1]

    def stage_idx(b):
        pltpu.sync_copy(x_hbm.at[pl.ds(b * xrows_per_b + wid * kmax, kmax)],
                        idx_all.at[pl.ds(b * kmax, kmax)])

    def out_slice(cc):
        k = cc % kmax
        b = cc // kmax
        return out_hbm.at[pl.ds(b * seq + wid * pos_per_w + k * _CHUNK,
                                _CHUNK)]

    def g_start(cc, par):
        pltpu.async_copy(wtab_hbm.at[idx_all.at[cc]], w[par], gsem[par])

    def g_wait(cc, par):
        pltpu.make_async_copy(
            wtab_hbm.at[idx_all.at[cc]], w[par], gsem[par]).wait()

    def o_start(cc, par):
        pltpu.async_copy(ob[par], out_slice(cc), osem[par])

    def o_wait(cc, par):
        pltpu.make_async_copy(ob[par], out_slice(cc), osem[par]).wait()

    def compute(cc, par):
        k = cc % kmax
        prow = k * _CHUNK

        def do_row(i, _, wb=w[par], obc=ob[par]):
            @plsc.parallel_loop(0, jcols, unroll=4)
            def do_j(j):
                sl = pl.ds(j * _LANES, _LANES)
                obc[i, sl] = wb[i, sl] * _SCALE + posbuf[prow + i, sl]
            return 0

        lax.fori_loop(0, _CHUNK, do_row, 0)

    stage_idx(0)
    pdesc = pltpu.async_copy(
        ptab_hbm.at[pl.ds(wid * pos_per_w, pos_per_w)], posbuf, psem)
    g_start(0, 0)
    g_start(1, 1)
    for b in range(1, batch):
        stage_idx(b)
    pdesc.wait()

    for cc in (0, 1):
        par = cc % 2
        g_wait(cc, par)
        compute(cc, par)
        o_start(cc, par)
        g_start(cc + 2, par)

    def t_body(t, _):
        for par in range(2):
            cc = 2 * t + par
            g_wait(cc, par)
            o_wait(cc - 2, par)
            compute(cc, par)
            o_start(cc, par)
            g_start(cc + 2, par)
        return 0

    lax.fori_loop(1, n_chunks // 2 - 1, t_body, 0)

    for cc in (n_chunks - 2, n_chunks - 1):
        par = cc % 2
        g_wait(cc, par)
        o_wait(cc - 2, par)
        compute(cc, par)
        o_start(cc, par)

    for cc in (n_chunks - 2, n_chunks - 1):
        o_wait(cc, cc % 2)


def kernel(x, word_emb, pos_emb):
    batch, seq = x.shape
    nrows = batch * seq

    mesh = plsc.VectorSubcoreMesh(core_axis_name="c", subcore_axis_name="s")
    body = functools.partial(_emb_body, batch=batch, seq=seq)
    out = pl.kernel(
        body,
        out_type=jax.ShapeDtypeStruct((nrows, _D), jnp.float32),
        mesh=mesh,
        scratch_types=[
            pltpu.VMEM((batch * (seq // _NW // _CHUNK), _CHUNK), jnp.int32),
            pltpu.VMEM((seq // _NW, _D), jnp.float32),
            pltpu.VMEM((_CHUNK, _D), jnp.float32),
            pltpu.VMEM((_CHUNK, _D), jnp.float32),
            pltpu.VMEM((_CHUNK, _D), jnp.float32),
            pltpu.VMEM((_CHUNK, _D), jnp.float32),
        ] + [pltpu.SemaphoreType.DMA] * 5,
    )(x.reshape(nrows // _CHUNK, _CHUNK).astype(jnp.int32), word_emb, pos_emb)
    return out.reshape(batch, seq, _D)


# speedup vs baseline: 1.1253x; 1.1253x over previous
import functools
import math

import jax
import jax.numpy as jnp
from jax import lax
from jax.experimental import pallas as pl
from jax.experimental.pallas import tpu as pltpu
from jax.experimental.pallas import tpu_sc as plsc

_D = 1024
_LANES = 16
_NC = 2
_NS = 16
_NW = _NC * _NS
_SCALE = math.sqrt(_D)
_CHUNK = 16


def _emb_body(x_hbm, wtab_hbm, ptab_hbm, out_hbm,
              idx_all, w0, w1, w2, pb0, pb1, ob0, ob1,
              g0, g1, g2, q0, q1, o0, o1, isem,
              *, batch, seq):
    wid = lax.axis_index("s") * _NC + lax.axis_index("c")
    pos_per_w = seq // _NW
    kmax = pos_per_w // _CHUNK
    n_chunks = kmax * batch
    jcols = _D // _LANES
    xrows_per_b = seq // _CHUNK

    w = [w0, w1, w2]
    pb = [pb0, pb1]
    ob = [ob0, ob1]
    gsem = [g0, g1, g2]
    psem = [q0, q1]
    osem = [o0, o1]

    def start_gather(cc):
        k, b = cc // batch, cc % batch
        return pltpu.async_copy(
            wtab_hbm.at[idx_all.at[b, k]], w[cc % 3], gsem[cc % 3])

    def start_pos(k):
        return pltpu.async_copy(
            ptab_hbm.at[pl.ds(wid * pos_per_w + k * _CHUNK, _CHUNK)],
            pb[k % 2], psem[k % 2])

    gdesc = [None] * n_chunks
    pdesc = [None] * kmax
    odesc = [None] * n_chunks
    idesc = pltpu.async_copy(
        x_hbm.at[pl.ds(0, batch), pl.ds(wid * kmax, kmax)], idx_all, isem)
    pdesc[0] = start_pos(0)
    if kmax > 1:
        pdesc[1] = start_pos(1)
    idesc.wait()
    gdesc[0] = start_gather(0)
    if n_chunks > 1:
        gdesc[1] = start_gather(1)

    for cc in range(n_chunks):
        k, b = cc // batch, cc % batch
        if cc + 2 < n_chunks:
            gdesc[cc + 2] = start_gather(cc + 2)
        if b == 0 and k >= 1 and k + 1 < kmax:
            pdesc[k + 1] = start_pos(k + 1)
        gdesc[cc].wait()
        if b == 0:
            pdesc[k].wait()
        if cc - 2 >= 0:
            odesc[cc - 2].wait()

        def do_row(i, _, wb=w[cc % 3], pbk=pb[k % 2], obc=ob[cc % 2]):
            @plsc.parallel_loop(0, jcols, unroll=4)
            def do_j(j):
                sl = pl.ds(j * _LANES, _LANES)
                obc[i, sl] = wb[i, sl] * _SCALE + pbk[i, sl]
            return 0

        lax.fori_loop(0, _CHUNK, do_row, 0)

        row0 = b * seq + wid * pos_per_w + k * _CHUNK
        odesc[cc] = pltpu.async_copy(
            ob[cc % 2], out_hbm.at[pl.ds(row0, _CHUNK)], osem[cc % 2])

    for cc in range(max(0, n_chunks - 2), n_chunks):
        odesc[cc].wait()


def kernel(x, word_emb, pos_emb):
    batch, seq = x.shape
    nrows = batch * seq

    mesh = plsc.VectorSubcoreMesh(core_axis_name="c", subcore_axis_name="s")
    body = functools.partial(_emb_body, batch=batch, seq=seq)
    out = pl.kernel(
        body,
        out_type=jax.ShapeDtypeStruct((nrows, _D), jnp.float32),
        mesh=mesh,
        scratch_types=[
            pltpu.VMEM((batch, seq // _NW // _CHUNK, _CHUNK), jnp.int32),
            pltpu.VMEM((_CHUNK, _D), jnp.float32),
            pltpu.VMEM((_CHUNK, _D), jnp.float32),
            pltpu.VMEM((_CHUNK, _D), jnp.float32),
            pltpu.VMEM((_CHUNK, _D), jnp.float32),
            pltpu.VMEM((_CHUNK, _D), jnp.float32),
            pltpu.VMEM((_CHUNK, _D), jnp.float32),
            pltpu.VMEM((_CHUNK, _D), jnp.float32),
        ] + [pltpu.SemaphoreType.DMA] * 8,
    )(x.reshape(batch, seq // _CHUNK, _CHUNK).astype(jnp.int32),
      word_emb, pos_emb)
    return out.reshape(batch, seq, _D)


# speedup vs baseline: 1.1544x; 1.0259x over previous
import functools
import math

import jax
import jax.numpy as jnp
from jax import lax
from jax.experimental import pallas as pl
from jax.experimental.pallas import tpu as pltpu
from jax.experimental.pallas import tpu_sc as plsc

_D = 1024
_LANES = 16
_NC = 2
_NS = 16
_NW = _NC * _NS
_SCALE = math.sqrt(_D)
_CHUNK = 16


def _emb_body(x_hbm, wtab_hbm, ptab_hbm, out_hbm,
              idx_all, w0, w1, w2, pb0, pb1, ob0, ob1,
              g0, g1, g2, q0, q1, o0, o1, isem,
              *, batch, seq):
    wid = lax.axis_index("s") * _NC + lax.axis_index("c")
    pos_per_w = seq // _NW
    kmax = pos_per_w // _CHUNK
    n_chunks = kmax * batch
    jcols = _D // _LANES
    xrows_per_b = seq // _CHUNK

    w = [w0, w1, w2]
    pb = [pb0, pb1]
    ob = [ob0, ob1]
    gsem = [g0, g1, g2]
    psem = [q0, q1]
    osem = [o0, o1]

    def start_gather(cc):
        k, b = cc // batch, cc % batch
        return pltpu.async_copy(
            wtab_hbm.at[idx_all.at[b, k]], w[cc % 3], gsem[cc % 3])

    def start_pos(k):
        return pltpu.async_copy(
            ptab_hbm.at[pl.ds(wid * pos_per_w + k * _CHUNK, _CHUNK)],
            pb[k % 2], psem[k % 2])

    gdesc = [None] * n_chunks
    pdesc = [None] * kmax
    odesc = [None] * n_chunks
    pltpu.sync_copy(x_hbm.at[0, pl.ds(wid * kmax, kmax)], idx_all.at[0])
    pdesc[0] = start_pos(0)
    gdesc[0] = start_gather(0)
    if n_chunks > 1:
        pltpu.sync_copy(x_hbm.at[1, pl.ds(wid * kmax, kmax)], idx_all.at[1])
        gdesc[1] = start_gather(1)
    idesc = None
    if batch > 2:
        idesc = pltpu.async_copy(
            x_hbm.at[pl.ds(2, batch - 2), pl.ds(wid * kmax, kmax)],
            idx_all.at[pl.ds(2, batch - 2)], isem)
    if kmax > 1:
        pdesc[1] = start_pos(1)
    if idesc is not None:
        idesc.wait()

    for cc in range(n_chunks):
        k, b = cc // batch, cc % batch
        if cc + 2 < n_chunks:
            gdesc[cc + 2] = start_gather(cc + 2)
        if b == 0 and k >= 1 and k + 1 < kmax:
            pdesc[k + 1] = start_pos(k + 1)
        gdesc[cc].wait()
        if b == 0:
            pdesc[k].wait()
        if cc - 2 >= 0:
            odesc[cc - 2].wait()

        def do_row(i, _, wb=w[cc % 3], pbk=pb[k % 2], obc=ob[cc % 2]):
            @plsc.parallel_loop(0, jcols, unroll=4)
            def do_j(j):
                sl = pl.ds(j * _LANES, _LANES)
                obc[i, sl] = wb[i, sl] * _SCALE + pbk[i, sl]
            return 0

        lax.fori_loop(0, _CHUNK, do_row, 0)

        row0 = b * seq + wid * pos_per_w + k * _CHUNK
        odesc[cc] = pltpu.async_copy(
            ob[cc % 2], out_hbm.at[pl.ds(row0, _CHUNK)], osem[cc % 2])

    for cc in range(max(0, n_chunks - 2), n_chunks):
        odesc[cc].wait()


def kernel(x, word_emb, pos_emb):
    batch, seq = x.shape
    nrows = batch * seq

    mesh = plsc.VectorSubcoreMesh(core_axis_name="c", subcore_axis_name="s")
    body = functools.partial(_emb_body, batch=batch, seq=seq)
    out = pl.kernel(
        body,
        out_type=jax.ShapeDtypeStruct((nrows, _D), jnp.float32),
        mesh=mesh,
        scratch_types=[
            pltpu.VMEM((batch, seq // _NW // _CHUNK, _CHUNK), jnp.int32),
            pltpu.VMEM((_CHUNK, _D), jnp.float32),
            pltpu.VMEM((_CHUNK, _D), jnp.float32),
            pltpu.VMEM((_CHUNK, _D), jnp.float32),
            pltpu.VMEM((_CHUNK, _D), jnp.float32),
            pltpu.VMEM((_CHUNK, _D), jnp.float32),
            pltpu.VMEM((_CHUNK, _D), jnp.float32),
            pltpu.VMEM((_CHUNK, _D), jnp.float32),
        ] + [pltpu.SemaphoreType.DMA] * 8,
    )(x.reshape(batch, seq // _CHUNK, _CHUNK).astype(jnp.int32),
      word_emb, pos_emb)
    return out.reshape(batch, seq, _D)


# speedup vs baseline: 1.1562x; 1.0015x over previous
import functools
import math

import jax
import jax.numpy as jnp
from jax import lax
from jax.experimental import pallas as pl
from jax.experimental.pallas import tpu as pltpu
from jax.experimental.pallas import tpu_sc as plsc

_D = 1024
_LANES = 16
_NC = 2
_NS = 16
_NW = _NC * _NS
_SCALE = math.sqrt(_D)
_CHUNK = 16


def _emb_body(x_hbm, wtab_hbm, ptab_hbm, out_hbm,
              idx_all, w0, w1, w2, pb0, pb1, ob0, ob1,
              g0, g1, g2, q0, q1, o0, o1, isem,
              *, batch, seq):
    wid = lax.axis_index("s") * _NC + lax.axis_index("c")
    pos_per_w = seq // _NW
    kmax = pos_per_w // _CHUNK
    n_chunks = kmax * batch
    jcols = _D // _LANES

    w = [w0, w1, w2]
    pb = [pb0, pb1]
    ob = [ob0, ob1]
    gsem = [g0, g1, g2]
    psem = [q0, q1]
    osem = [o0, o1]

    def start_gather(cc):
        k, b = cc // batch, cc % batch
        return pltpu.async_copy(
            wtab_hbm.at[idx_all.at[b, k]], w[cc % 3], gsem[cc % 3])

    def start_pos(k):
        return pltpu.async_copy(
            ptab_hbm.at[pl.ds(wid * pos_per_w + k * _CHUNK, _CHUNK)],
            pb[k % 2], psem[k % 2])

    gdesc = [None] * n_chunks
    pdesc = [None] * kmax
    odesc = [None] * n_chunks
    pltpu.sync_copy(x_hbm.at[0, pl.ds(wid * kmax, kmax)], idx_all.at[0])
    pdesc[0] = start_pos(0)
    gdesc[0] = start_gather(0)
    if n_chunks > 1:
        pltpu.sync_copy(x_hbm.at[1, pl.ds(wid * kmax, kmax)], idx_all.at[1])
        gdesc[1] = start_gather(1)
    idesc = None
    if batch > 2:
        idesc = pltpu.async_copy(
            x_hbm.at[pl.ds(2, batch - 2), pl.ds(wid * kmax, kmax)],
            idx_all.at[pl.ds(2, batch - 2)], isem)
    if kmax > 1:
        pdesc[1] = start_pos(1)
    if idesc is not None:
        idesc.wait()

    for cc in range(n_chunks):
        k, b = cc // batch, cc % batch
        if cc + 2 < n_chunks:
            gdesc[cc + 2] = start_gather(cc + 2)
        if b == 0 and k >= 1 and k + 1 < kmax:
            pdesc[k + 1] = start_pos(k + 1)
        gdesc[cc].wait()
        if b == 0:
            pdesc[k].wait()
        if cc - 2 >= 0:
            odesc[cc - 2].wait()

        def do_row(i, _, wb=w[cc % 3], pbk=pb[k % 2], obc=ob[cc % 2]):
            @plsc.parallel_loop(0, jcols, unroll=4)
            def do_j(j):
                sl = pl.ds(j * _LANES, _LANES)
                obc[i, sl] = wb[i, sl] * _SCALE + pbk[i, sl]
            return 0

        lax.fori_loop(0, _CHUNK, do_row, 0)

        row0 = b * seq + wid * pos_per_w + k * _CHUNK
        odesc[cc] = pltpu.async_copy(
            ob[cc % 2], out_hbm.at[pl.ds(row0, _CHUNK)], osem[cc % 2])

    for cc in range(max(0, n_chunks - 2), n_chunks):
        odesc[cc].wait()


def kernel(x, word_emb, pos_emb):
    batch, seq = x.shape
    nrows = batch * seq

    mesh = plsc.VectorSubcoreMesh(core_axis_name="c", subcore_axis_name="s")
    body = functools.partial(_emb_body, batch=batch, seq=seq)
    out = pl.kernel(
        body,
        out_type=jax.ShapeDtypeStruct((nrows, _D), jnp.float32),
        mesh=mesh,
        scratch_types=[
            pltpu.VMEM((batch, seq // _NW // _CHUNK, _CHUNK), jnp.int32),
            pltpu.VMEM((_CHUNK, _D), jnp.float32),
            pltpu.VMEM((_CHUNK, _D), jnp.float32),
            pltpu.VMEM((_CHUNK, _D), jnp.float32),
            pltpu.VMEM((_CHUNK, _D), jnp.float32),
            pltpu.VMEM((_CHUNK, _D), jnp.float32),
            pltpu.VMEM((_CHUNK, _D), jnp.float32),
            pltpu.VMEM((_CHUNK, _D), jnp.float32),
        ] + [pltpu.SemaphoreType.DMA] * 8,
    )(x.reshape(batch, seq // _CHUNK, _CHUNK).astype(jnp.int32),
      word_emb, pos_emb)
    return out.reshape(batch, seq, _D)
